# baseline (speedup 1.0000x reference)
import functools

import jax
import jax.numpy as jnp
from jax import lax
from jax.experimental import pallas as pl
from jax.experimental.pallas import tpu as pltpu
from jax.experimental.pallas import tpu_sc as plsc

NC = 2
NS = 16
NW = NC * NS
NBUF = 4


@functools.partial(jax.jit, static_argnames=("n", "g", "d"))
def _embed(x, lut, *, n, g, d):
    rpw = n // NW
    scale = jnp.sqrt(jnp.float32(d))

    mesh = plsc.VectorSubcoreMesh(
        core_axis_name="c", subcore_axis_name="s",
        num_cores=NC, num_subcores=NS)

    def body(idx_hbm, lut_hbm, out_hbm, idx_scr, ins, outs, sins, souts):
        wid = lax.axis_index("s") * NC + lax.axis_index("c")
        rbase = wid * rpw
        pltpu.sync_copy(idx_hbm.at[pl.ds(rbase, rpw)], idx_scr)

        def start_gather(r, b):
            pltpu.async_copy(lut_hbm.at[idx_scr.at[r]], ins[b], sins[b])

        def wait_gather(r, b):
            pltpu.make_async_copy(
                lut_hbm.at[idx_scr.at[r]], ins[b], sins[b]).wait()

        def start_scatter(r, b):
            pltpu.async_copy(outs[b], out_hbm.at[rbase + r], souts[b])

        def wait_scatter(r, b):
            pltpu.make_async_copy(
                outs[b], out_hbm.at[rbase + r], souts[b]).wait()

        def scale_row(b):
            src, dst = ins[b], outs[b]

            @plsc.parallel_loop(0, g, unroll=4)
            def _(i):
                for j in range(d // 16):
                    sl = pl.ds(j * 16, 16)
                    dst[i, sl] = src[i, sl] * scale

        def do_row(r, b, first, last):
            wait_gather(r, b)
            if not first:
                wait_scatter(r, b)
            scale_row(b)
            if not last:
                start_gather(r + NBUF, b)
            start_scatter(r, b)

        for b in range(NBUF):
            start_gather(b, b)
        for b in range(NBUF):
            do_row(b, b, True, False)
        nr = rpw // NBUF

        @pl.loop(1, nr - 1)
        def _(rr):
            for b in range(NBUF):
                do_row(rr * NBUF + b, b, False, False)

        for b in range(NBUF):
            do_row((nr - 1) * NBUF + b, b, False, True)
        for b in range(NBUF):
            wait_scatter((nr - 1) * NBUF + b, b)

    f32 = jnp.float32
    run = pl.kernel(
        body,
        out_type=jax.ShapeDtypeStruct((n, g, d), f32),
        mesh=mesh,
        scratch_types=[
            pltpu.VMEM((rpw, g), jnp.int32),
            tuple(pltpu.VMEM((g, d), f32) for _ in range(NBUF)),
            tuple(pltpu.VMEM((g, d), f32) for _ in range(NBUF)),
            tuple(pltpu.SemaphoreType.DMA for _ in range(NBUF)),
            tuple(pltpu.SemaphoreType.DMA for _ in range(NBUF)),
        ],
        compiler_params=pltpu.CompilerParams(use_tc_tiling_on_sc=False),
    )
    return run(x, lut)


def kernel(x, lut):
    n, g = x.shape
    d = lut.shape[1]
    assert n % NW == 0 and (n // NW) % NBUF == 0, (n, NW, NBUF)
    if x.dtype != jnp.int32:
        x = x.astype(jnp.int32)
    return _embed(x, lut, n=n, g=g, d=d)
